# baseline (speedup 1.0000x reference)
import jax
import jax.numpy as jnp
from jax.experimental import pallas as pl
from jax.experimental.pallas import tpu as pltpu

N = 10000
F = 512
BM = 400
NB = N // BM
BX = 2000
NS = N // BX


def _dot(a, b):
    return jnp.dot(a, b, preferred_element_type=jnp.float32)


def _mixed_dot(a, b):
    return jax.lax.dot_general(
        a, b,
        dimension_numbers=(((1,), (0,)), ((), ())),
        preferred_element_type=jnp.float32,
    )


def _body(x_ref, adj_ref, w1_ref, b1_ref, b2_ref, w2_ref,
          o_ref, s1_ref, s2_ref):
    i = pl.program_id(0)

    @pl.when(i < NS)
    def _small_mm():
        s1_ref[pl.ds(i * BX, BX), :] = _dot(
            x_ref[...], w1_ref[...]).astype(jnp.bfloat16)

    @pl.when(jnp.logical_and(i >= NS, i < NS + NB))
    def _layer1():
        acc = _mixed_dot(adj_ref[...], s1_ref[...])
        act = jnp.tanh(acc + b1_ref[...])
        s2_ref[pl.ds(((i - NS) % NB) * BM, BM), :] = _dot(
            act, w2_ref[...]).astype(jnp.bfloat16)

    @pl.when(i >= NS + NB)
    def _layer2():
        acc = _mixed_dot(adj_ref[...], s2_ref[...])
        o_ref[...] = jnp.tanh(acc + b2_ref[...])


def kernel(x, adj, W1, b1, W2, b2):
    return pl.pallas_call(
        _body,
        grid=(NS + 2 * NB,),
        in_specs=[
            pl.BlockSpec((BX, F), lambda i: (jnp.minimum(i, NS - 1), 0)),
            pl.BlockSpec((BM, N),
                         lambda i: (jnp.where(i >= NS + NB,
                                              NS + 2 * NB - 1 - i,
                                              jnp.maximum(i - NS, 0)), 0)),
            pl.BlockSpec((F, F), lambda i: (0, 0)),
            pl.BlockSpec((1, F), lambda i: (0, 0)),
            pl.BlockSpec((1, F), lambda i: (0, 0)),
            pl.BlockSpec((F, F), lambda i: (0, 0)),
        ],
        out_specs=pl.BlockSpec(
            (BM, F),
            lambda i: (jnp.clip(NS + 2 * NB - 1 - i, 0, NB - 1), 0)),
        out_shape=jax.ShapeDtypeStruct((N, F), jnp.float32),
        scratch_shapes=[
            pltpu.VMEM((N, F), jnp.bfloat16),
            pltpu.VMEM((N, F), jnp.bfloat16),
        ],
        compiler_params=pltpu.CompilerParams(
            dimension_semantics=("arbitrary",),
            vmem_limit_bytes=64 * 1024 * 1024,
        ),
    )(x, adj, W1, b1.reshape(1, F), b2.reshape(1, F), W2)


# speedup vs baseline: 1.0028x; 1.0028x over previous
import jax
import jax.numpy as jnp
from jax.experimental import pallas as pl
from jax.experimental.pallas import tpu as pltpu

N = 10000
F = 512
BM = 400
NB = N // BM
BX = 2000
NS = N // BX


def _dot(a, b):
    return jnp.dot(a, b, preferred_element_type=jnp.float32)


def _mixed_dot(a, b):
    return jax.lax.dot_general(
        a, b,
        dimension_numbers=(((1,), (0,)), ((), ())),
        preferred_element_type=jnp.float32,
    )


def _body(x_ref, adj_ref, w1_ref, b1_ref, b2_ref, w2_ref,
          o_ref, s1_ref, s2_ref):
    i = pl.program_id(0)

    @pl.when(i < NS)
    def _small_mm():
        s1_ref[pl.ds(i * BX, BX), :] = _dot(
            x_ref[...], w1_ref[...]).astype(jnp.bfloat16)

    @pl.when(jnp.logical_and(i >= NS, i < NS + NB))
    def _layer1():
        acc = _dot(adj_ref[...].astype(jnp.bfloat16), s1_ref[...])
        act = jnp.tanh(acc + b1_ref[...])
        s2_ref[pl.ds(((i - NS) % NB) * BM, BM), :] = _dot(
            act, w2_ref[...]).astype(jnp.bfloat16)

    @pl.when(i >= NS + NB)
    def _layer2():
        acc = _dot(adj_ref[...].astype(jnp.bfloat16), s2_ref[...])
        o_ref[...] = jnp.tanh(acc + b2_ref[...])


def kernel(x, adj, W1, b1, W2, b2):
    return pl.pallas_call(
        _body,
        grid=(NS + 2 * NB,),
        in_specs=[
            pl.BlockSpec((BX, F), lambda i: (jnp.minimum(i, NS - 1), 0)),
            pl.BlockSpec((BM, N),
                         lambda i: (jnp.where(i >= NS + NB,
                                              NS + 2 * NB - 1 - i,
                                              jnp.maximum(i - NS, 0)), 0)),
            pl.BlockSpec((F, F), lambda i: (0, 0)),
            pl.BlockSpec((1, F), lambda i: (0, 0)),
            pl.BlockSpec((1, F), lambda i: (0, 0)),
            pl.BlockSpec((F, F), lambda i: (0, 0)),
        ],
        out_specs=pl.BlockSpec(
            (BM, F),
            lambda i: (jnp.clip(NS + 2 * NB - 1 - i, 0, NB - 1), 0)),
        out_shape=jax.ShapeDtypeStruct((N, F), jnp.float32),
        scratch_shapes=[
            pltpu.VMEM((N, F), jnp.bfloat16),
            pltpu.VMEM((N, F), jnp.bfloat16),
        ],
        compiler_params=pltpu.CompilerParams(
            dimension_semantics=("arbitrary",),
            vmem_limit_bytes=64 * 1024 * 1024,
        ),
    )(x, adj, W1, b1.reshape(1, F), b2.reshape(1, F), W2)


# speedup vs baseline: 1.0034x; 1.0007x over previous
import jax
import jax.numpy as jnp
from jax.experimental import pallas as pl
from jax.experimental.pallas import tpu as pltpu

N = 10000
F = 512
BM = 400
NB = N // BM
BX = 2000
NS = N // BX


def _dot(a, b):
    return jnp.dot(a, b, preferred_element_type=jnp.float32)


def _mixed_dot(a, b):
    return jax.lax.dot_general(
        a, b,
        dimension_numbers=(((1,), (0,)), ((), ())),
        preferred_element_type=jnp.float32,
    )


def _body(x_ref, adj_ref, w1_ref, b1_ref, b2_ref, w2_ref,
          o_ref, s1_ref, s2_ref):
    i = pl.program_id(0)

    @pl.when(i < NS)
    def _small_mm():
        s1_ref[pl.ds(i * BX, BX), :] = _dot(
            x_ref[...], w1_ref[...]).astype(jnp.bfloat16)

    @pl.when(jnp.logical_and(i >= NS, i < NS + NB))
    def _layer1():
        acc = _mixed_dot(adj_ref[...], s1_ref[...])
        act = jnp.tanh(acc + b1_ref[...])
        s2_ref[pl.ds(((i - NS) % NB) * BM, BM), :] = _dot(
            act, w2_ref[...]).astype(jnp.bfloat16)

    @pl.when(i >= NS + NB)
    def _layer2():
        acc = _mixed_dot(adj_ref[...], s2_ref[...])
        o_ref[...] = jnp.tanh(acc + b2_ref[...])


def kernel(x, adj, W1, b1, W2, b2):
    return pl.pallas_call(
        _body,
        grid=(NS + 2 * NB,),
        in_specs=[
            pl.BlockSpec((BX, F), lambda i: (jnp.minimum(i, NS - 1), 0)),
            pl.BlockSpec((BM, N),
                         lambda i: (jnp.where(i >= NS + NB,
                                              NS + 2 * NB - 1 - i,
                                              jnp.maximum(i - NS, 0)), 0)),
            pl.BlockSpec((F, F), lambda i: (0, 0)),
            pl.BlockSpec((1, F), lambda i: (0, 0)),
            pl.BlockSpec((1, F), lambda i: (0, 0)),
            pl.BlockSpec((F, F), lambda i: (0, 0)),
        ],
        out_specs=pl.BlockSpec(
            (BM, F),
            lambda i: (jnp.clip(NS + 2 * NB - 1 - i, 0, NB - 1), 0)),
        out_shape=jax.ShapeDtypeStruct((N, F), jnp.float32),
        scratch_shapes=[
            pltpu.VMEM((N, F), jnp.bfloat16),
            pltpu.VMEM((N, F), jnp.bfloat16),
        ],
        compiler_params=pltpu.CompilerParams(
            dimension_semantics=("arbitrary",),
            vmem_limit_bytes=64 * 1024 * 1024,
        ),
    )(x, adj, W1, b1.reshape(1, F), b2.reshape(1, F), W2)
